# baseline (speedup 1.0000x reference)
import functools

import jax
import jax.numpy as jnp
from jax import lax
from jax.experimental import pallas as pl
from jax.experimental.pallas import tpu as pltpu
from jax.experimental.pallas import tpu_sc as plsc

_NUM_CORES = 2
_NUM_SUBCORES = 16
_NW = _NUM_CORES * _NUM_SUBCORES
_CHUNK = 16
_NBUF = 4


@functools.lru_cache(maxsize=None)
def _build_gather(B, D):
    b_per_w = B // _NW
    n_chunks = b_per_w // _CHUNK
    n_groups = n_chunks // _NBUF
    assert n_chunks % _NBUF == 0 and n_groups >= 2
    mesh = plsc.VectorSubcoreMesh(core_axis_name="c", subcore_axis_name="s")

    @functools.partial(
        pl.kernel,
        mesh=mesh,
        out_type=jax.ShapeDtypeStruct((B, D), jnp.float32),
        scratch_types=[
            pltpu.VMEM((n_chunks, _CHUNK), jnp.int32),
            *[pltpu.VMEM((_CHUNK, D), jnp.float32) for _ in range(_NBUF)],
            *[pltpu.SemaphoreType.DMA for _ in range(2 * _NBUF)],
        ],
    )
    def gather_kernel(table_hbm, idx_hbm, out_hbm, idx_v, *rest):
        bufs = rest[:_NBUF]
        gsem = rest[_NBUF:2 * _NBUF]
        wsem = rest[2 * _NBUF:]
        wid = lax.axis_index("s") * _NUM_CORES + lax.axis_index("c")
        base = wid * b_per_w
        pltpu.sync_copy(idx_hbm.at[wid], idx_v)

        def gather_start(b, c):
            pltpu.async_copy(table_hbm.at[idx_v.at[c]], bufs[b], gsem[b])

        def gather_wait(b, c):
            pltpu.make_async_copy(
                table_hbm.at[idx_v.at[c]], bufs[b], gsem[b]
            ).wait()

        def write_start(b, c):
            pltpu.async_copy(
                bufs[b], out_hbm.at[pl.ds(base + c * _CHUNK, _CHUNK)], wsem[b]
            )

        def write_wait(b, c):
            pltpu.make_async_copy(
                bufs[b], out_hbm.at[pl.ds(base + c * _CHUNK, _CHUNK)], wsem[b]
            ).wait()

        for b in range(_NBUF - 1):
            gather_start(b, b)

        def body(b, c, first=False):
            gather_wait(b, c)
            write_start(b, c)
            b3 = (b + _NBUF - 1) % _NBUF
            if not first:
                write_wait(b3, c - 1)
            gather_start(b3, c + _NBUF - 1)

        for b in range(_NBUF):
            body(b, b, first=(b == 0))

        def group(g, carry):
            for b in range(_NBUF):
                body(b, g * _NBUF + b)
            return carry

        lax.fori_loop(1, n_groups - 1, group, 0)

        for b in range(_NBUF):
            c = (n_groups - 1) * _NBUF + b
            gather_wait(b, c)
            write_start(b, c)
            if b == 0:
                write_wait(_NBUF - 1, c - 1)
                gather_start(_NBUF - 1, c + _NBUF - 1)
        for b in range(_NBUF):
            write_wait(b, (n_groups - 1) * _NBUF + b)

    return gather_kernel


def kernel(input_ids, weight):
    orig_shape = input_ids.shape
    D = weight.shape[1]
    B = input_ids.size
    idx = input_ids.reshape(_NW, (B // _NW) // _CHUNK, _CHUNK).astype(jnp.int32)
    out = _build_gather(B, D)(weight.astype(jnp.float32), idx)
    return out.reshape(*orig_shape, D)


# speedup vs baseline: 1.0028x; 1.0028x over previous
import functools

import jax
import jax.numpy as jnp
from jax import lax
from jax.experimental import pallas as pl
from jax.experimental.pallas import tpu as pltpu
from jax.experimental.pallas import tpu_sc as plsc

_NUM_CORES = 2
_NUM_SUBCORES = 16
_NW = _NUM_CORES * _NUM_SUBCORES
_CHUNK = 32
_NBUF = 3


@functools.lru_cache(maxsize=None)
def _build_gather(batch, seq, D):
    B = batch * seq
    b_per_w = B // _NW
    n_chunks = b_per_w // _CHUNK
    segs_per_row = seq // b_per_w
    assert b_per_w % _CHUNK == 0 and seq % b_per_w == 0
    assert n_chunks >= 2 * _NBUF
    mesh = plsc.VectorSubcoreMesh(core_axis_name="c", subcore_axis_name="s")

    @functools.partial(
        pl.kernel,
        mesh=mesh,
        out_type=jax.ShapeDtypeStruct((B, D), jnp.float32),
        scratch_types=[
            pltpu.VMEM((b_per_w,), jnp.int32),
            *[pltpu.VMEM((_CHUNK, D), jnp.float32) for _ in range(_NBUF)],
            *[pltpu.SemaphoreType.DMA for _ in range(2 * _NBUF)],
        ],
    )
    def gather_kernel(table_hbm, idx_hbm, out_hbm, idx_v, *rest):
        bufs = rest[:_NBUF]
        gsem = rest[_NBUF:2 * _NBUF]
        wsem = rest[2 * _NBUF:]
        wid = lax.axis_index("s") * _NUM_CORES + lax.axis_index("c")
        base = wid * b_per_w
        pltpu.sync_copy(
            idx_hbm.at[
                wid // segs_per_row,
                pl.ds((wid % segs_per_row) * b_per_w, b_per_w),
            ],
            idx_v,
        )

        def gather_start(b, c):
            pltpu.async_copy(
                table_hbm.at[idx_v.at[pl.ds(c * _CHUNK, _CHUNK)]], bufs[b], gsem[b]
            )

        def gather_wait(b, c):
            pltpu.make_async_copy(
                table_hbm.at[idx_v.at[pl.ds(c * _CHUNK, _CHUNK)]], bufs[b], gsem[b]
            ).wait()

        def write_start(b, c):
            pltpu.async_copy(
                bufs[b], out_hbm.at[pl.ds(base + c * _CHUNK, _CHUNK)], wsem[b]
            )

        def write_wait(b, c):
            pltpu.make_async_copy(
                bufs[b], out_hbm.at[pl.ds(base + c * _CHUNK, _CHUNK)], wsem[b]
            ).wait()

        def body(b, c, first=False):
            gather_wait(b, c)
            write_start(b, c)
            b3 = (b + _NBUF - 1) % _NBUF
            if not first:
                write_wait(b3, c - 1)
            gather_start(b3, c + _NBUF - 1)

        for b in range(_NBUF - 1):
            gather_start(b, b)

        for c0 in range(_NBUF):
            body(c0 % _NBUF, c0, first=(c0 == 0))

        n_main = (n_chunks - 2 * _NBUF + 1) // _NBUF

        def group(g, carry):
            for j in range(_NBUF):
                c = _NBUF + g * _NBUF + j
                body(j % _NBUF, c)
            return carry

        lax.fori_loop(0, n_main, group, 0)

        for c0 in range(_NBUF + n_main * _NBUF, n_chunks - _NBUF + 1):
            body(c0 % _NBUF, c0)
        for c0 in range(n_chunks - _NBUF + 1, n_chunks):
            gather_wait(c0 % _NBUF, c0)
            write_start(c0 % _NBUF, c0)
        for c0 in range(n_chunks - _NBUF, n_chunks):
            write_wait(c0 % _NBUF, c0)

    return gather_kernel


def kernel(input_ids, weight):
    batch, seq = input_ids.shape
    D = weight.shape[1]
    if input_ids.dtype != jnp.int32:
        input_ids = input_ids.astype(jnp.int32)
    out = _build_gather(batch, seq, D)(weight.astype(jnp.float32), input_ids)
    return out.reshape(batch, seq, D)


# speedup vs baseline: 1.0100x; 1.0072x over previous
import functools

import jax
import jax.numpy as jnp
from jax import lax
from jax.experimental import pallas as pl
from jax.experimental.pallas import tpu as pltpu
from jax.experimental.pallas import tpu_sc as plsc

_NUM_CORES = 2
_NUM_SUBCORES = 16
_NW = _NUM_CORES * _NUM_SUBCORES
_CHUNK = 16
_NBUF = 6


@functools.lru_cache(maxsize=None)
def _build_gather(batch, seq, D):
    B = batch * seq
    b_per_w = B // _NW
    n_chunks = b_per_w // _CHUNK
    segs_per_row = seq // b_per_w
    assert b_per_w % _CHUNK == 0 and seq % b_per_w == 0
    assert n_chunks >= 2 * _NBUF
    mesh = plsc.VectorSubcoreMesh(core_axis_name="c", subcore_axis_name="s")

    @functools.partial(
        pl.kernel,
        mesh=mesh,
        out_type=jax.ShapeDtypeStruct((B, D), jnp.float32),
        scratch_types=[
            pltpu.VMEM((b_per_w,), jnp.int32),
            *[pltpu.VMEM((_CHUNK, D), jnp.float32) for _ in range(_NBUF)],
            *[pltpu.SemaphoreType.DMA for _ in range(2 * _NBUF)],
        ],
    )
    def gather_kernel(table_hbm, idx_hbm, out_hbm, idx_v, *rest):
        bufs = rest[:_NBUF]
        gsem = rest[_NBUF:2 * _NBUF]
        wsem = rest[2 * _NBUF:]
        wid = lax.axis_index("s") * _NUM_CORES + lax.axis_index("c")
        base = wid * b_per_w
        pltpu.sync_copy(
            idx_hbm.at[
                wid // segs_per_row,
                pl.ds((wid % segs_per_row) * b_per_w, b_per_w),
            ],
            idx_v,
        )

        def gather_start(b, c):
            pltpu.async_copy(
                table_hbm.at[idx_v.at[pl.ds(c * _CHUNK, _CHUNK)]], bufs[b], gsem[b]
            )

        def gather_wait(b, c):
            pltpu.make_async_copy(
                table_hbm.at[idx_v.at[pl.ds(c * _CHUNK, _CHUNK)]], bufs[b], gsem[b]
            ).wait()

        def write_start(b, c):
            pltpu.async_copy(
                bufs[b], out_hbm.at[pl.ds(base + c * _CHUNK, _CHUNK)], wsem[b]
            )

        def write_wait(b, c):
            pltpu.make_async_copy(
                bufs[b], out_hbm.at[pl.ds(base + c * _CHUNK, _CHUNK)], wsem[b]
            ).wait()

        def body(b, c, first=False):
            gather_wait(b, c)
            write_start(b, c)
            b3 = (b + _NBUF - 1) % _NBUF
            if not first:
                write_wait(b3, c - 1)
            gather_start(b3, c + _NBUF - 1)

        for b in range(_NBUF - 1):
            gather_start(b, b)

        for c0 in range(_NBUF):
            body(c0 % _NBUF, c0, first=(c0 == 0))

        n_main = (n_chunks - 2 * _NBUF + 1) // _NBUF

        def group(g, carry):
            for j in range(_NBUF):
                c = _NBUF + g * _NBUF + j
                body(j % _NBUF, c)
            return carry

        lax.fori_loop(0, n_main, group, 0)

        for c0 in range(_NBUF + n_main * _NBUF, n_chunks - _NBUF + 1):
            body(c0 % _NBUF, c0)
        for c0 in range(n_chunks - _NBUF + 1, n_chunks):
            gather_wait(c0 % _NBUF, c0)
            write_start(c0 % _NBUF, c0)
        for c0 in range(n_chunks - _NBUF, n_chunks):
            write_wait(c0 % _NBUF, c0)

    return gather_kernel


def kernel(input_ids, weight):
    batch, seq = input_ids.shape
    D = weight.shape[1]
    if input_ids.dtype != jnp.int32:
        input_ids = input_ids.astype(jnp.int32)
    out = _build_gather(batch, seq, D)(weight.astype(jnp.float32), input_ids)
    return out.reshape(batch, seq, D)
